# baseline (speedup 1.0000x reference)
import functools

import jax
import jax.numpy as jnp
from jax import lax
from jax.experimental import pallas as pl
from jax.experimental.pallas import tpu as pltpu
from jax.experimental.pallas import tpu_sc as plsc

N_GRAPHS = 1024
LANES = 16
N_SUBCORES = 16
UNROLL = 8


@functools.lru_cache(maxsize=None)
def _build_sc_call(n_nodes, n_species, n_graphs):
    chunk = -(-n_nodes // (N_SUBCORES * 8)) * 8
    n_vregs = -(-chunk // LANES)
    chunk = n_vregs * LANES
    n_pad_vregs = -(-n_vregs // UNROLL) * UNROLL
    pad_n = n_pad_vregs * LANES
    last = N_SUBCORES - 1
    valid_last = n_nodes - last * chunk
    fill_last = valid_last - valid_last % LANES
    w_pad = -(-n_species // LANES) * LANES
    acc_len = -(-(n_graphs + 1) // LANES) * LANES
    span = 128
    n_comb = n_graphs // span
    span_vregs = span // LANES

    mesh = plsc.VectorSubcoreMesh(
        core_axis_name="c", subcore_axis_name="s", num_cores=1
    )

    @functools.partial(
        pl.kernel,
        out_type=jax.ShapeDtypeStruct((n_graphs,), jnp.float32),
        mesh=mesh,
        compiler_params=pltpu.CompilerParams(needs_layout_passes=False),
        scratch_types=[
            pltpu.VMEM((pad_n,), jnp.int32),
            pltpu.VMEM((pad_n + LANES,), jnp.int32),
            pltpu.VMEM((w_pad,), jnp.float32),
            pltpu.VMEM((acc_len,), jnp.float32),
            pltpu.VMEM((LANES,), jnp.int32),
            pltpu.VMEM((N_SUBCORES, LANES), jnp.int32),
            pltpu.VMEM((N_SUBCORES, span), jnp.float32),
            pltpu.VMEM((span,), jnp.float32),
            pltpu.VMEM_SHARED((N_SUBCORES, LANES), jnp.int32),
            pltpu.VMEM_SHARED((N_SUBCORES, acc_len), jnp.float32),
        ],
    )
    def sc_call(idx_hbm, bat_hbm, w_hbm, out_hbm,
                idx_v, bat_v, w_v, acc_v, min_v, gmin_v, comb_v, out_v,
                mins_sh, acc_sh):
        tid = lax.axis_index("s")
        base = tid * chunk
        pad_i = jnp.full((LANES,), n_species - 1, jnp.int32)
        pad_b = jnp.full((LANES,), n_graphs, jnp.int32)

        def fill_tail(valid, fill_from):
            for j in range(fill_from, pad_n + LANES, LANES):
                if j + LANES <= valid:
                    continue
                if j < valid:
                    lo = lax.iota(jnp.int32, LANES) + (j - valid)
                    keep = lo < 0
                    idx_v[pl.ds(j, LANES)] = jnp.where(
                        keep, idx_v[pl.ds(j, LANES)], pad_i)
                    bat_v[pl.ds(j, LANES)] = jnp.where(
                        keep, bat_v[pl.ds(j, LANES)], pad_b)
                else:
                    if j < pad_n:
                        idx_v[pl.ds(j, LANES)] = pad_i
                    bat_v[pl.ds(j, LANES)] = pad_b

        @pl.when(tid < last)
        def _():
            pltpu.sync_copy(idx_hbm.at[pl.ds(base, chunk)], idx_v.at[pl.ds(0, chunk)])
            pltpu.sync_copy(bat_hbm.at[pl.ds(base, chunk)], bat_v.at[pl.ds(0, chunk)])
            pltpu.sync_copy(w_hbm, w_v.at[pl.ds(0, n_species)])
            fill_tail(chunk, chunk)

        @pl.when(tid == last)
        def _():
            pltpu.sync_copy(idx_hbm.at[pl.ds(base, valid_last)],
                            idx_v.at[pl.ds(0, valid_last)])
            pltpu.sync_copy(bat_hbm.at[pl.ds(base, valid_last)],
                            bat_v.at[pl.ds(0, valid_last)])
            pltpu.sync_copy(w_hbm, w_v.at[pl.ds(0, n_species)])
            fill_tail(valid_last, fill_last)

        def min_body(jo, m):
            for u in range(UNROLL):
                m = jnp.minimum(
                    m, idx_v[pl.ds((jo * UNROLL + u) * LANES, LANES)])
            return m

        m = lax.fori_loop(
            0, n_pad_vregs // UNROLL, min_body,
            jnp.full((LANES,), 2**30, jnp.int32),
        )
        min_v[...] = m
        pltpu.sync_copy(min_v, mins_sh.at[tid])

        def zero_body(j, carry):
            acc_v[pl.ds(j * LANES, LANES)] = jnp.zeros((LANES,), jnp.float32)
            return carry

        lax.fori_loop(0, acc_len // LANES, zero_body, 0)
        plsc.subcore_barrier()

        pltpu.sync_copy(mins_sh, gmin_v)

        def gmin_body(k, mm):
            return jnp.minimum(mm, gmin_v[k, :])

        mm = lax.fori_loop(
            0, N_SUBCORES, gmin_body, jnp.full((LANES,), 2**30, jnp.int32),
        )
        gmin = mm[0]
        for k in range(1, LANES):
            gmin = jnp.minimum(gmin, mm[k])
        shift = jnp.where(gmin >= 1, jnp.int32(1), jnp.int32(0))

        lane = lax.iota(jnp.int32, LANES)
        last_lane = lane == (LANES - 1)
        not_last = lane < (LANES - 1)

        def main_body(jo, carry):
            for u in range(UNROLL):
                j = jo * UNROLL + u
                b = bat_v[pl.ds(j * LANES, LANES)]
                i = idx_v[pl.ds(j * LANES, LANES)]
                i = jnp.maximum(i - shift, 0)
                v = plsc.load_gather(w_v, [i])
                plsc.addupdate_scatter(acc_v, [b], v)
            return carry

        lax.fori_loop(0, n_pad_vregs // UNROLL, main_body, 0)

        pltpu.sync_copy(acc_v, acc_sh.at[tid])
        plsc.subcore_barrier()

        @pl.when(tid < n_comb)
        def _():
            pltpu.sync_copy(acc_sh.at[:, pl.ds(tid * span, span)], comb_v)

            def comb_body(k, carry):
                return tuple(
                    carry[c] + comb_v[k, pl.ds(c * LANES, LANES)]
                    for c in range(span_vregs)
                )

            ss = lax.fori_loop(
                0, N_SUBCORES, comb_body,
                tuple(jnp.zeros((LANES,), jnp.float32)
                      for _ in range(span_vregs)),
            )
            for c in range(span_vregs):
                out_v[pl.ds(c * LANES, LANES)] = ss[c]
            pltpu.sync_copy(out_v, out_hbm.at[pl.ds(tid * span, span)])

    return sc_call


@jax.jit
def kernel(node_feats, batch, ref_weight):
    n_nodes = node_feats.shape[0]
    n_species, out_dim = ref_weight.shape
    idx = node_feats[:, 0].astype(jnp.int32)
    sc_call = _build_sc_call(n_nodes, n_species, N_GRAPHS)
    out = sc_call(idx, batch.astype(jnp.int32), ref_weight[:, 0])
    return out.reshape(N_GRAPHS, out_dim)


# speedup vs baseline: 1.0021x; 1.0021x over previous
import functools

import jax
import jax.numpy as jnp
from jax import lax
from jax.experimental import pallas as pl
from jax.experimental.pallas import tpu as pltpu
from jax.experimental.pallas import tpu_sc as plsc

N_GRAPHS = 1024
LANES = 16
N_SUBCORES = 16
UNROLL = 8


@functools.lru_cache(maxsize=None)
def _build_sc_call(n_nodes, n_species, n_graphs):
    chunk = -(-n_nodes // (N_SUBCORES * 8)) * 8
    n_vregs = -(-chunk // LANES)
    chunk = n_vregs * LANES
    terr = -(-n_vregs // UNROLL) * UNROLL
    pad_n = terr * LANES
    last = N_SUBCORES - 1
    valid_last = n_nodes - last * chunk
    fill_last = valid_last - valid_last % LANES
    w_pad = -(-n_species // LANES) * LANES
    acc_len = -(-(n_graphs + 1) // LANES) * LANES
    span = 128
    n_comb = n_graphs // span
    span_vregs = span // LANES

    mesh = plsc.VectorSubcoreMesh(
        core_axis_name="c", subcore_axis_name="s", num_cores=1
    )

    @functools.partial(
        pl.kernel,
        out_type=jax.ShapeDtypeStruct((n_graphs,), jnp.float32),
        mesh=mesh,
        compiler_params=pltpu.CompilerParams(needs_layout_passes=False),
        scratch_types=[
            pltpu.VMEM((pad_n,), jnp.int32),
            pltpu.VMEM((pad_n,), jnp.int32),
            pltpu.VMEM((w_pad,), jnp.float32),
            pltpu.VMEM((acc_len,), jnp.float32),
            pltpu.VMEM((LANES,), jnp.int32),
            pltpu.VMEM((2 * LANES,), jnp.int32),
            pltpu.VMEM((N_SUBCORES, LANES), jnp.int32),
            pltpu.VMEM((N_SUBCORES, span), jnp.float32),
            pltpu.VMEM((span,), jnp.float32),
            pltpu.VMEM_SHARED((N_SUBCORES, LANES), jnp.int32),
            pltpu.VMEM_SHARED((N_SUBCORES, acc_len), jnp.float32),
        ],
    )
    def sc_call(idx_hbm, bat_hbm, w_hbm, out_hbm,
                idx_v, bat_v, w_v, acc_v, min_v, drain_v, gmin_v, comb_v,
                out_v, mins_sh, acc_sh):
        tid = lax.axis_index("s")
        base = tid * chunk
        pad_i = jnp.full((LANES,), n_species - 1, jnp.int32)
        pad_b = jnp.full((LANES,), n_graphs, jnp.int32)

        def fill_tail(valid, fill_from):
            for j in range(fill_from, pad_n, LANES):
                if j + LANES <= valid:
                    continue
                if j < valid:
                    lo = lax.iota(jnp.int32, LANES) + (j - valid)
                    keep = lo < 0
                    idx_v[pl.ds(j, LANES)] = jnp.where(
                        keep, idx_v[pl.ds(j, LANES)], pad_i)
                    bat_v[pl.ds(j, LANES)] = jnp.where(
                        keep, bat_v[pl.ds(j, LANES)], pad_b)
                else:
                    idx_v[pl.ds(j, LANES)] = pad_i
                    bat_v[pl.ds(j, LANES)] = pad_b

        @pl.when(tid < last)
        def _():
            pltpu.sync_copy(idx_hbm.at[pl.ds(base, chunk)],
                            idx_v.at[pl.ds(0, chunk)])
            pltpu.sync_copy(bat_hbm.at[pl.ds(base, chunk)],
                            bat_v.at[pl.ds(0, chunk)])
            pltpu.sync_copy(w_hbm, w_v.at[pl.ds(0, n_species)])
            fill_tail(chunk, chunk)

        @pl.when(tid == last)
        def _():
            pltpu.sync_copy(idx_hbm.at[pl.ds(base, valid_last)],
                            idx_v.at[pl.ds(0, valid_last)])
            pltpu.sync_copy(bat_hbm.at[pl.ds(base, valid_last)],
                            bat_v.at[pl.ds(0, valid_last)])
            pltpu.sync_copy(w_hbm, w_v.at[pl.ds(0, n_species)])
            fill_tail(valid_last, fill_last)

        def min_body(jo, m):
            for u in range(UNROLL):
                m = jnp.minimum(
                    m, idx_v[pl.ds((jo * UNROLL + u) * LANES, LANES)])
            return m

        m = lax.fori_loop(
            0, terr // UNROLL, min_body,
            jnp.full((LANES,), 2**30, jnp.int32),
        )
        min_v[...] = m
        pltpu.sync_copy(min_v, mins_sh.at[tid])

        def zero_body(j, carry):
            acc_v[pl.ds(j * LANES, LANES)] = jnp.zeros((LANES,), jnp.float32)
            return carry

        lax.fori_loop(0, acc_len // LANES, zero_body, 0)
        plsc.subcore_barrier()

        pltpu.sync_copy(mins_sh, gmin_v)

        def gmin_body(k, mm):
            return jnp.minimum(mm, gmin_v[k, :])

        mm = lax.fori_loop(
            0, N_SUBCORES, gmin_body, jnp.full((LANES,), 2**30, jnp.int32),
        )
        gmin = mm[0]
        for k in range(1, LANES):
            gmin = jnp.minimum(gmin, mm[k])
        shift = jnp.where(gmin >= 1, jnp.int32(1), jnp.int32(0))

        lane = lax.iota(jnp.int32, LANES)
        last_lane = lane == (LANES - 1)
        not_last = lane < (LANES - 1)
        zero_f = jnp.zeros((LANES,), jnp.float32)

        sent = jnp.full((LANES,), n_graphs, jnp.int32)

        def main_body(to, carry):
            pos, prev_b, acc = carry
            for _ in range(UNROLL):
                i = plsc.load_gather(idx_v, [pos])
                b = plsc.load_gather(bat_v, [pos])
                ii = jnp.maximum(i - shift, 0)
                v = plsc.load_gather(w_v, [ii])
                chg = b != prev_b
                plsc.addupdate_scatter(
                    acc_v,
                    [jnp.where(chg, prev_b, sent)],
                    jnp.where(chg, acc, zero_f),
                )
                acc = jnp.where(chg, zero_f, acc) + v
                prev_b = b
                pos = pos + 1
            return pos, prev_b, acc

        pos0 = lane * terr
        pos, prev_b, acc = lax.fori_loop(
            0, terr // UNROLL, main_body, (pos0, pad_b, zero_f)
        )

        plsc.addupdate_scatter(acc_v, [prev_b], acc)

        pltpu.sync_copy(acc_v, acc_sh.at[tid])
        plsc.subcore_barrier()

        @pl.when(tid < n_comb)
        def _():
            pltpu.sync_copy(acc_sh.at[:, pl.ds(tid * span, span)], comb_v)

            def comb_body(k, carry):
                return tuple(
                    carry[c] + comb_v[k, pl.ds(c * LANES, LANES)]
                    for c in range(span_vregs)
                )

            ss = lax.fori_loop(
                0, N_SUBCORES, comb_body,
                tuple(jnp.zeros((LANES,), jnp.float32)
                      for _ in range(span_vregs)),
            )
            for c in range(span_vregs):
                out_v[pl.ds(c * LANES, LANES)] = ss[c]
            pltpu.sync_copy(out_v, out_hbm.at[pl.ds(tid * span, span)])

    return sc_call


@jax.jit
def kernel(node_feats, batch, ref_weight):
    n_nodes = node_feats.shape[0]
    n_species, out_dim = ref_weight.shape
    idx = node_feats[:, 0].astype(jnp.int32)
    sc_call = _build_sc_call(n_nodes, n_species, N_GRAPHS)
    out = sc_call(idx, batch.astype(jnp.int32), ref_weight[:, 0])
    return out.reshape(N_GRAPHS, out_dim)


# speedup vs baseline: 1.1832x; 1.1808x over previous
import functools

import jax
import jax.numpy as jnp
from jax import lax
from jax.experimental import pallas as pl
from jax.experimental.pallas import tpu as pltpu
from jax.experimental.pallas import tpu_sc as plsc

N_GRAPHS = 1024
LANES = 16
N_SUBCORES = 16
UNROLL = 8
N_REGIONS = 4


@functools.lru_cache(maxsize=None)
def _build_sc_call(n_nodes, n_species, n_graphs):
    chunk = -(-n_nodes // (N_SUBCORES * 8)) * 8
    terr = -(-chunk // LANES)
    if terr % 2 == 0:
        terr += 1
    pad_n = terr * LANES
    chunk = min(chunk, n_nodes)
    last = N_SUBCORES - 1
    valid_last = n_nodes - last * chunk
    fill_last = valid_last - valid_last % LANES
    w_pad = -(-n_species // LANES) * LANES
    rstride = n_graphs + 17
    acc16_len = -(-(N_REGIONS * rstride) // LANES) * LANES
    acc_len = n_graphs
    span = 128
    n_comb = n_graphs // span
    span_vregs = span // LANES

    mesh = plsc.VectorSubcoreMesh(
        core_axis_name="c", subcore_axis_name="s", num_cores=1
    )

    @functools.partial(
        pl.kernel,
        out_type=jax.ShapeDtypeStruct((n_graphs,), jnp.float32),
        mesh=mesh,
        compiler_params=pltpu.CompilerParams(needs_layout_passes=False),
        scratch_types=[
            pltpu.VMEM((pad_n,), jnp.int32),
            pltpu.VMEM((pad_n,), jnp.int32),
            pltpu.VMEM((w_pad,), jnp.float32),
            pltpu.VMEM((w_pad,), jnp.float32),
            pltpu.VMEM((acc16_len,), jnp.float32),
            pltpu.VMEM((acc_len,), jnp.float32),
            pltpu.VMEM((LANES,), jnp.int32),
            pltpu.VMEM((N_SUBCORES, LANES), jnp.int32),
            pltpu.VMEM((N_SUBCORES, span), jnp.float32),
            pltpu.VMEM((span,), jnp.float32),
            pltpu.VMEM_SHARED((N_SUBCORES, LANES), jnp.int32),
            pltpu.VMEM_SHARED((N_SUBCORES, acc_len), jnp.float32),
        ],
    )
    def sc_call(idx_hbm, bat_hbm, w_hbm, out_hbm,
                idx_v, bat_v, w_v, w_eff, acc16_v, acc_v, min_v, gmin_v,
                comb_v, out_v, mins_sh, acc_sh):
        tid = lax.axis_index("s")
        base = tid * chunk
        pad_i = jnp.full((LANES,), n_species - 1, jnp.int32)
        pad_b = jnp.full((LANES,), n_graphs, jnp.int32)
        lane = lax.iota(jnp.int32, LANES)

        def fill_tail(valid, fill_from):
            for j in range(fill_from, pad_n, LANES):
                if j + LANES <= valid:
                    continue
                if j < valid:
                    keep = (lane + (j - valid)) < 0
                    idx_v[pl.ds(j, LANES)] = jnp.where(
                        keep, idx_v[pl.ds(j, LANES)], pad_i)
                    bat_v[pl.ds(j, LANES)] = jnp.where(
                        keep, bat_v[pl.ds(j, LANES)], pad_b)
                else:
                    idx_v[pl.ds(j, LANES)] = pad_i
                    bat_v[pl.ds(j, LANES)] = pad_b

        @pl.when(tid < last)
        def _():
            pltpu.sync_copy(idx_hbm.at[pl.ds(base, chunk)],
                            idx_v.at[pl.ds(0, chunk)])
            pltpu.sync_copy(bat_hbm.at[pl.ds(base, chunk)],
                            bat_v.at[pl.ds(0, chunk)])
            pltpu.sync_copy(w_hbm, w_v.at[pl.ds(0, n_species)])
            fill_tail(chunk, chunk - chunk % LANES)

        @pl.when(tid == last)
        def _():
            pltpu.sync_copy(idx_hbm.at[pl.ds(base, valid_last)],
                            idx_v.at[pl.ds(0, valid_last)])
            pltpu.sync_copy(bat_hbm.at[pl.ds(base, valid_last)],
                            bat_v.at[pl.ds(0, valid_last)])
            pltpu.sync_copy(w_hbm, w_v.at[pl.ds(0, n_species)])
            fill_tail(valid_last, fill_last)

        n_mv = pad_n // LANES

        def min_body(jo, m):
            for u in range(UNROLL):
                m = jnp.minimum(
                    m, idx_v[pl.ds((jo * UNROLL + u) * LANES, LANES)])
            return m

        m = jnp.full((LANES,), 2**30, jnp.int32)
        m = lax.fori_loop(0, n_mv // UNROLL, min_body, m)
        for j in range((n_mv // UNROLL) * UNROLL, n_mv):
            m = jnp.minimum(m, idx_v[pl.ds(j * LANES, LANES)])
        min_v[...] = m
        pltpu.sync_copy(min_v, mins_sh.at[tid])

        def zero_body(j, carry):
            acc16_v[pl.ds(j * LANES, LANES)] = jnp.zeros((LANES,), jnp.float32)
            return carry

        lax.fori_loop(0, acc16_len // LANES, zero_body, 0)
        plsc.subcore_barrier()

        pltpu.sync_copy(mins_sh, gmin_v)

        def gmin_body(k, mm):
            return jnp.minimum(mm, gmin_v[k, :])

        mm = lax.fori_loop(
            0, N_SUBCORES, gmin_body, jnp.full((LANES,), 2**30, jnp.int32),
        )
        gmin = mm[0]
        for k in range(1, LANES):
            gmin = jnp.minimum(gmin, mm[k])
        shift = jnp.where(gmin >= 1, jnp.int32(1), jnp.int32(0))

        for k in range(w_pad // LANES):
            src = jnp.maximum(lane + (k * LANES) - shift, 0)
            w_eff[pl.ds(k * LANES, LANES)] = plsc.load_gather(w_v, [src])

        reg_off = (lane % N_REGIONS) * rstride

        def step(pos):
            i = plsc.load_gather(idx_v, [pos])
            b = plsc.load_gather(bat_v, [pos])
            v = plsc.load_gather(w_eff, [i])
            plsc.addupdate_scatter(acc16_v, [b + reg_off], v)
            return pos + 1

        def main_body(to, pos):
            for _ in range(UNROLL):
                pos = step(pos)
            return pos

        pos = lane * terr
        pos = lax.fori_loop(0, terr // UNROLL, main_body, pos)
        for _ in range(terr % UNROLL):
            pos = step(pos)

        def merge_body(j, carry):
            s = acc16_v[pl.ds(j * LANES, LANES)]
            for r in range(1, N_REGIONS):
                s = s + acc16_v[pl.ds(r * rstride + j * LANES, LANES)]
            acc_v[pl.ds(j * LANES, LANES)] = s
            return carry

        lax.fori_loop(0, acc_len // LANES, merge_body, 0)

        pltpu.sync_copy(acc_v, acc_sh.at[tid])
        plsc.subcore_barrier()

        @pl.when(tid < n_comb)
        def _():
            pltpu.sync_copy(acc_sh.at[:, pl.ds(tid * span, span)], comb_v)

            def comb_body(k, carry):
                return tuple(
                    carry[c] + comb_v[k, pl.ds(c * LANES, LANES)]
                    for c in range(span_vregs)
                )

            ss = lax.fori_loop(
                0, N_SUBCORES, comb_body,
                tuple(jnp.zeros((LANES,), jnp.float32)
                      for _ in range(span_vregs)),
            )
            for c in range(span_vregs):
                out_v[pl.ds(c * LANES, LANES)] = ss[c]
            pltpu.sync_copy(out_v, out_hbm.at[pl.ds(tid * span, span)])

    return sc_call


@jax.jit
def kernel(node_feats, batch, ref_weight):
    n_nodes = node_feats.shape[0]
    n_species, out_dim = ref_weight.shape
    idx = node_feats[:, 0].astype(jnp.int32)
    sc_call = _build_sc_call(n_nodes, n_species, N_GRAPHS)
    out = sc_call(idx, batch.astype(jnp.int32), ref_weight[:, 0])
    return out.reshape(N_GRAPHS, out_dim)
